# Initial kernel scaffold; baseline (speedup 1.0000x reference)
import jax
import jax.numpy as jnp
from jax.experimental import pallas as pl


def kernel(input, weights):
    raise NotImplementedError("write your pallas kernel here")


# speedup vs baseline: 2.3112x; 2.3112x over previous
import functools

import jax
import jax.numpy as jnp
from jax import lax
from jax.experimental import pallas as pl
from jax.experimental.pallas import tpu as pltpu
from jax.experimental.pallas import tpu_sc as plsc

PAD = 1


def _positions_body(tok_ref, pos_ref):
    tok = tok_ref[...]
    mask = (tok != PAD).astype(jnp.int32)
    n = tok.shape[1]
    col = lax.broadcasted_iota(jnp.int32, tok.shape, 1)
    acc = mask
    sh = 1
    while sh < n:
        rolled = pltpu.roll(acc, sh, 1)
        acc = acc + jnp.where(col >= sh, rolled, 0)
        sh *= 2
    pos_ref[...] = acc * mask + PAD


def _positions(tokens):
    return pl.pallas_call(
        _positions_body,
        out_shape=jax.ShapeDtypeStruct(tokens.shape, jnp.int32),
    )(tokens)


@functools.partial(jax.jit, static_argnums=(2, 3, 4))
def _sc_gather(table, idx, B, D, CH):
    info = plsc.get_sparse_core_info()
    NC, NS = info.num_cores, info.num_subcores
    NW = NC * NS
    b_per_w = B // NW
    n_ch = b_per_w // CH
    assert n_ch % 2 == 0
    mesh = plsc.VectorSubcoreMesh(core_axis_name="c", subcore_axis_name="s")

    @functools.partial(
        pl.kernel,
        mesh=mesh,
        out_type=jax.ShapeDtypeStruct((B, D), jnp.float32),
        scratch_types=[
            pltpu.VMEM((b_per_w,), jnp.int32),
            pltpu.VMEM((CH, D), jnp.float32),
            pltpu.VMEM((CH, D), jnp.float32),
            pltpu.SemaphoreType.DMA,
            pltpu.SemaphoreType.DMA,
        ],
    )
    def k(table_hbm, idx_hbm, out_hbm, idx_v, buf0, buf1, gs0, gs1):
        wid = lax.axis_index("s") * NC + lax.axis_index("c")
        base = wid * b_per_w
        pltpu.sync_copy(idx_hbm.at[pl.ds(base, b_per_w)], idx_v)

        def fire(g, buf, sem):
            pltpu.async_copy(table_hbm.at[idx_v.at[pl.ds(g * CH, CH)]], buf, sem)

        def wait(g, buf, sem):
            pltpu.make_async_copy(
                table_hbm.at[idx_v.at[pl.ds(g * CH, CH)]], buf, sem
            ).wait()

        def put(g, buf):
            pltpu.sync_copy(buf, out_hbm.at[pl.ds(base + g * CH, CH)])

        fire(0, buf0, gs0)

        def body(i, carry):
            g0 = i * 2
            g1 = g0 + 1
            fire(g1, buf1, gs1)
            wait(g0, buf0, gs0)
            put(g0, buf0)

            @pl.when(g1 + 1 < n_ch)
            def _():
                fire(g1 + 1, buf0, gs0)

            wait(g1, buf1, gs1)
            put(g1, buf1)
            return carry

        lax.fori_loop(0, n_ch // 2, body, 0)

    return k(table, idx)


def kernel(input, weights):
    bsz, seq_len = input.shape
    tokens = input.astype(jnp.int32)
    positions = _positions(tokens)
    D = weights.shape[1]
    B = bsz * seq_len
    out = _sc_gather(weights, positions.reshape(B), B, D, 32)
    return out.reshape(bsz, seq_len, D)


# speedup vs baseline: 2.3381x; 1.0117x over previous
import functools

import jax
import jax.numpy as jnp
from jax import lax
from jax.experimental import pallas as pl
from jax.experimental.pallas import tpu as pltpu
from jax.experimental.pallas import tpu_sc as plsc

PAD = 1


@functools.partial(jax.jit, static_argnums=(2, 3, 4))
def _sc_embed(tokens_flat, table, B, D, CH):
    info = plsc.get_sparse_core_info()
    NC, NS, L = info.num_cores, info.num_subcores, info.num_lanes
    NW = NC * NS
    b_per_w = B // NW
    n_ch = b_per_w // CH
    n_vec = b_per_w // L
    slabs_per_row = 8
    mesh = plsc.VectorSubcoreMesh(core_axis_name="c", subcore_axis_name="s")

    _dnums = lax.GatherDimensionNumbers(
        offset_dims=(), collapsed_slice_dims=(0,), start_index_map=(0,))

    def _vgather(x, idx):
        return lax.gather(x, idx[:, None], _dnums, (1,),
                          mode=lax.GatherScatterMode.PROMISE_IN_BOUNDS)

    def _csum16(x):
        lanes_c = lax.iota(jnp.int32, L)
        for sh in (1, 2, 4, 8):
            rolled = _vgather(x, jnp.maximum(lanes_c - sh, 0))
            x = x + jnp.where(lanes_c >= sh, rolled, 0)
        return x

    def _last_splat(x):
        return _vgather(x, jnp.zeros((L,), jnp.int32) + (L - 1))

    @functools.partial(
        pl.kernel,
        mesh=mesh,
        out_type=jax.ShapeDtypeStruct((B, D), jnp.float32),
        scratch_types=[
            pltpu.VMEM((b_per_w,), jnp.int32),
            pltpu.VMEM((b_per_w,), jnp.int32),
            pltpu.VMEM((L,), jnp.int32),
            pltpu.VMEM((NS, L), jnp.int32),
            pltpu.VMEM_SHARED((NS, L), jnp.int32),
            pltpu.VMEM((CH, D), jnp.float32),
            pltpu.VMEM((CH, D), jnp.float32),
            pltpu.SemaphoreType.DMA,
            pltpu.SemaphoreType.DMA,
        ],
    )
    def k(tok_hbm, table_hbm, out_hbm, tok_v, idx_v, my_cnt_v, cnt_all_v,
          cnt_sh, buf0, buf1, gs0, gs1):
        c = lax.axis_index("c")
        s = lax.axis_index("s")
        row = 2 * c + s // slabs_per_row
        slab = s % slabs_per_row
        base = row * (slabs_per_row * b_per_w) + slab * b_per_w

        pltpu.sync_copy(tok_hbm.at[pl.ds(base, b_per_w)], tok_v)

        def cbody(j, acc):
            t = tok_v[pl.ds(j * L, L)]
            return acc + jnp.where(t != PAD, 1, 0).astype(jnp.int32)

        acc = lax.fori_loop(0, n_vec, cbody, jnp.zeros((L,), jnp.int32))
        cnt_splat = _last_splat(_csum16(acc))

        my_cnt_v[...] = cnt_splat
        pltpu.sync_copy(my_cnt_v, cnt_sh.at[s])
        plsc.subcore_barrier()
        pltpu.sync_copy(cnt_sh, cnt_all_v)
        lanes = lax.iota(jnp.int32, L)
        cvec = jnp.zeros((L,), jnp.int32)
        for j in range(NS):
            cvec = cvec + jnp.where(lanes == j, cnt_all_v[j], 0)
        in_row = (lanes >= (s - slab)) & (lanes < s)
        start = _last_splat(_csum16(jnp.where(in_row, cvec, 0)))

        def pbody(j, carry):
            t = tok_v[pl.ds(j * L, L)]
            v = jnp.where(t != PAD, 1, 0).astype(jnp.int32)
            cs = _csum16(v)
            idx_v[pl.ds(j * L, L)] = (cs + carry) * v + PAD
            return carry + _last_splat(cs)

        lax.fori_loop(0, n_vec, pbody, start)

        def fire(g, buf, sem):
            pltpu.async_copy(table_hbm.at[idx_v.at[pl.ds(g * CH, CH)]], buf, sem)

        def wait(g, buf, sem):
            pltpu.make_async_copy(
                table_hbm.at[idx_v.at[pl.ds(g * CH, CH)]], buf, sem
            ).wait()

        def put(g, buf):
            pltpu.sync_copy(buf, out_hbm.at[pl.ds(base + g * CH, CH)])

        fire(0, buf0, gs0)

        def body(i, carry):
            g0 = i * 2
            g1 = g0 + 1
            fire(g1, buf1, gs1)
            wait(g0, buf0, gs0)
            put(g0, buf0)

            @pl.when(g1 + 1 < n_ch)
            def _():
                fire(g1 + 1, buf0, gs0)

            wait(g1, buf1, gs1)
            put(g1, buf1)
            return carry

        lax.fori_loop(0, n_ch // 2, body, 0)

    return k(tokens_flat, table)


def kernel(input, weights):
    bsz, seq_len = input.shape
    tokens = input.astype(jnp.int32)
    D = weights.shape[1]
    B = bsz * seq_len
    out = _sc_embed(tokens.reshape(B), weights, B, D, 32)
    return out.reshape(bsz, seq_len, D)


# speedup vs baseline: 2.3430x; 1.0021x over previous
import functools

import jax
import jax.numpy as jnp
from jax import lax
from jax.experimental import pallas as pl
from jax.experimental.pallas import tpu as pltpu
from jax.experimental.pallas import tpu_sc as plsc

PAD = 1


@functools.partial(jax.jit, static_argnums=(2, 3, 4))
def _sc_embed(tokens_flat, table, B, D, CH):
    info = plsc.get_sparse_core_info()
    NC, NS, L = info.num_cores, info.num_subcores, info.num_lanes
    NW = NC * NS
    b_per_w = B // NW
    n_ch = b_per_w // CH
    n_vec = b_per_w // L
    slabs_per_row = 8
    mesh = plsc.VectorSubcoreMesh(core_axis_name="c", subcore_axis_name="s")

    _dnums = lax.GatherDimensionNumbers(
        offset_dims=(), collapsed_slice_dims=(0,), start_index_map=(0,))

    def _vgather(x, idx):
        return lax.gather(x, idx[:, None], _dnums, (1,),
                          mode=lax.GatherScatterMode.PROMISE_IN_BOUNDS)

    def _csum16(x):
        lanes_c = lax.iota(jnp.int32, L)
        for sh in (1, 2, 4, 8):
            rolled = _vgather(x, jnp.maximum(lanes_c - sh, 0))
            x = x + jnp.where(lanes_c >= sh, rolled, 0)
        return x

    def _last_splat(x):
        return _vgather(x, jnp.zeros((L,), jnp.int32) + (L - 1))

    @functools.partial(
        pl.kernel,
        mesh=mesh,
        out_type=jax.ShapeDtypeStruct((B, D), jnp.float32),
        scratch_types=[
            pltpu.VMEM((b_per_w,), jnp.int32),
            pltpu.VMEM((b_per_w,), jnp.int32),
            pltpu.VMEM((L,), jnp.int32),
            pltpu.VMEM((NS, L), jnp.int32),
            pltpu.VMEM_SHARED((NS, L), jnp.int32),
            pltpu.VMEM((CH, D), jnp.float32),
            pltpu.VMEM((CH, D), jnp.float32),
            pltpu.VMEM((CH, D), jnp.float32),
            pltpu.VMEM((CH, D), jnp.float32),
            pltpu.SemaphoreType.DMA,
            pltpu.SemaphoreType.DMA,
            pltpu.SemaphoreType.DMA,
            pltpu.SemaphoreType.DMA,
            pltpu.SemaphoreType.DMA,
            pltpu.SemaphoreType.DMA,
            pltpu.SemaphoreType.DMA,
            pltpu.SemaphoreType.DMA,
        ],
    )
    def k(tok_hbm, table_hbm, out_hbm, tok_v, idx_v, my_cnt_v, cnt_all_v,
          cnt_sh, buf0, buf1, buf2, buf3,
          gs0, gs1, gs2, gs3, ws0, ws1, ws2, ws3):
        c = lax.axis_index("c")
        s = lax.axis_index("s")
        row = 2 * c + s // slabs_per_row
        slab = s % slabs_per_row
        base = row * (slabs_per_row * b_per_w) + slab * b_per_w

        pltpu.sync_copy(tok_hbm.at[pl.ds(base, b_per_w)], tok_v)

        def cbody(j, acc):
            t = tok_v[pl.ds(j * L, L)]
            return acc + jnp.where(t != PAD, 1, 0).astype(jnp.int32)

        acc = lax.fori_loop(0, n_vec, cbody, jnp.zeros((L,), jnp.int32))
        cnt_splat = _last_splat(_csum16(acc))

        my_cnt_v[...] = cnt_splat
        pltpu.sync_copy(my_cnt_v, cnt_sh.at[s])
        plsc.subcore_barrier()
        pltpu.sync_copy(cnt_sh, cnt_all_v)
        lanes = lax.iota(jnp.int32, L)
        cvec = jnp.zeros((L,), jnp.int32)
        for j in range(NS):
            cvec = cvec + jnp.where(lanes == j, cnt_all_v[j], 0)
        in_row = (lanes >= (s - slab)) & (lanes < s)
        start = _last_splat(_csum16(jnp.where(in_row, cvec, 0)))

        def pbody(j, carry):
            t = tok_v[pl.ds(j * L, L)]
            v = jnp.where(t != PAD, 1, 0).astype(jnp.int32)
            cs = _csum16(v)
            idx_v[pl.ds(j * L, L)] = (cs + carry) * v + PAD
            return carry + _last_splat(cs)

        lax.fori_loop(0, n_vec, pbody, start)

        bufs = (buf0, buf1, buf2, buf3)
        gss = (gs0, gs1, gs2, gs3)
        wss = (ws0, ws1, ws2, ws3)
        NB = 4

        def fire(g, b):
            pltpu.async_copy(
                table_hbm.at[idx_v.at[pl.ds(g * CH, CH)]], bufs[b], gss[b])

        def wait_g(g, b):
            pltpu.make_async_copy(
                table_hbm.at[idx_v.at[pl.ds(g * CH, CH)]], bufs[b], gss[b]
            ).wait()

        def fire_w(g, b):
            pltpu.async_copy(bufs[b], out_hbm.at[pl.ds(base + g * CH, CH)],
                             wss[b])

        def wait_w(g, b):
            pltpu.make_async_copy(
                bufs[b], out_hbm.at[pl.ds(base + g * CH, CH)], wss[b]
            ).wait()

        fire(0, 0)
        fire(1, 1)

        def body(i, carry):
            for b in range(NB):
                g = i * NB + b
                wait_g(g, b)
                fire_w(g, b)
                nb = (b + 2) % NB

                @pl.when(g - 2 >= 0)
                def _():
                    wait_w(g - 2, nb)

                @pl.when(g + 2 < n_ch)
                def _():
                    fire(g + 2, nb)

            return carry

        lax.fori_loop(0, n_ch // NB, body, 0)
        wait_w(n_ch - 2, (n_ch - 2) % NB)
        wait_w(n_ch - 1, (n_ch - 1) % NB)

    return k(tokens_flat, table)


def kernel(input, weights):
    bsz, seq_len = input.shape
    tokens = input.astype(jnp.int32)
    D = weights.shape[1]
    B = bsz * seq_len
    out = _sc_embed(tokens.reshape(B), weights, B, D, 16)
    return out.reshape(bsz, seq_len, D)
